# Initial kernel scaffold; baseline (speedup 1.0000x reference)
import jax
import jax.numpy as jnp
from jax.experimental import pallas as pl


def kernel(x, spiral_adj, W, b):
    raise NotImplementedError("write your pallas kernel here")


# speedup vs baseline: 7.6374x; 7.6374x over previous
import functools

import jax
import jax.numpy as jnp
from jax import lax
from jax.experimental import pallas as pl
from jax.experimental.pallas import tpu as pltpu
from jax.experimental.pallas import tpu_sc as plsc

IN_C = 128
SPIRAL = 32
OUT_C = 128
N_PTS = 10000

_info = plsc.get_sparse_core_info()
NC = _info.num_cores
NS = _info.num_subcores
L = _info.num_lanes
NW = NC * NS

P = 8
ROWS = P * SPIRAL
NG = ROWS // 128
NCHUNK = N_PTS // P
VPR = OUT_C // L


def _mm_body(x_ref, w_ref, y_ref):
    y_ref[...] = jnp.dot(x_ref[...], w_ref[...],
                         preferred_element_type=jnp.float32)


def _matmul(x2d, wmat):
    BM = 400
    return pl.pallas_call(
        _mm_body,
        grid=(N_PTS // BM,),
        in_specs=[
            pl.BlockSpec((BM, IN_C), lambda i: (i, 0)),
            pl.BlockSpec((IN_C, SPIRAL * OUT_C), lambda i: (0, 0)),
        ],
        out_specs=pl.BlockSpec((BM, SPIRAL * OUT_C), lambda i: (i, 0)),
        out_shape=jax.ShapeDtypeStruct((N_PTS, SPIRAL * OUT_C), jnp.float32),
    )(x2d, wmat)


_mesh = plsc.VectorSubcoreMesh(core_axis_name="c", subcore_axis_name="s")


@functools.partial(
    pl.kernel,
    out_type=jax.ShapeDtypeStruct((N_PTS, OUT_C), jnp.float32),
    mesh=_mesh,
    scratch_types=[
        pltpu.VMEM((ROWS,), jnp.int32),
        pltpu.VMEM((NG, 128), jnp.int32),
        pltpu.VMEM((ROWS, OUT_C), jnp.float32),
        pltpu.VMEM((P, OUT_C), jnp.float32),
        pltpu.VMEM((OUT_C,), jnp.float32),
        pltpu.SemaphoreType.DMA,
    ],
)
def _sc_gather(y_hbm, adj_hbm, b_hbm, out_hbm,
               adj_v, idx_v, rows_v, out_v, bias_v, sem):
    wid = lax.axis_index("s") * NC + lax.axis_index("c")
    pltpu.sync_copy(b_hbm, bias_v)
    n_iter = (NCHUNK - wid + NW - 1) // NW

    def chunk_body(i, carry):
        c = wid + i * NW
        pltpu.sync_copy(adj_hbm.at[pl.ds(c * ROWS, ROWS)], adj_v)
        for j in range(NG):
            for v in range(128 // L):
                f0 = j * 128 + v * L
                a = adj_v[pl.ds(f0, L)]
                svec = lax.iota(jnp.int32, L) + (f0 % SPIRAL)
                idx_v[j, pl.ds(v * L, L)] = a * SPIRAL + svec
        cps = [
            pltpu.async_copy(y_hbm.at[idx_v.at[j]],
                             rows_v.at[pl.ds(j * 128, 128)], sem)
            for j in range(NG)
        ]
        for cp in cps:
            cp.wait()
        for p in range(P):
            accs = tuple(bias_v[pl.ds(v * L, L)] for v in range(VPR))

            def s_body(s, acc):
                r = p * SPIRAL + s
                return tuple(a + rows_v[r, pl.ds(v * L, L)]
                             for v, a in enumerate(acc))

            accs = lax.fori_loop(0, SPIRAL, s_body, accs)
            for v in range(VPR):
                z = accs[v]
                y = jnp.where(z > 0.0,
                              z, jnp.exp(jnp.minimum(z, 0.0)) - 1.0)
                out_v[p, pl.ds(v * L, L)] = y

            @pl.when(c * P + p == N_PTS - 1)
            def _():
                zero = jnp.zeros((L,), jnp.float32)
                for v in range(VPR):
                    out_v[p, pl.ds(v * L, L)] = zero
        pltpu.sync_copy(out_v, out_hbm.at[pl.ds(c * P, P)])
        return carry

    lax.fori_loop(0, n_iter, chunk_body, 0)


def kernel(x, spiral_adj, W, b):
    x2d = x.reshape(N_PTS, IN_C)
    adj = spiral_adj.reshape(N_PTS * SPIRAL).astype(jnp.int32)
    wmat = (W.reshape(OUT_C, SPIRAL, IN_C)
            .transpose(2, 1, 0)
            .reshape(IN_C, SPIRAL * OUT_C))
    y = _matmul(x2d, wmat)
    yr = y.reshape(N_PTS * SPIRAL, OUT_C)
    out2d = _sc_gather(yr, adj, b)
    return out2d.reshape(1, N_PTS, OUT_C)


# speedup vs baseline: 13.6875x; 1.7922x over previous
import functools

import jax
import jax.numpy as jnp
from jax import lax
from jax.experimental import pallas as pl
from jax.experimental.pallas import tpu as pltpu
from jax.experimental.pallas import tpu_sc as plsc

IN_C = 128
SPIRAL = 32
OUT_C = 128
N_PTS = 10000

_info = plsc.get_sparse_core_info()
NC = _info.num_cores
NS = _info.num_subcores
L = _info.num_lanes
NW = NC * NS

P = 8
ROWS = P * SPIRAL
NG = ROWS // 128
NCHUNK = N_PTS // P
VPR = OUT_C // L


def _mm_body(x_ref, w_ref, y_ref):
    y_ref[0] = jnp.dot(x_ref[...], w_ref[0],
                       preferred_element_type=jnp.float32)


def _matmul(x2d, wmat3):
    return pl.pallas_call(
        _mm_body,
        grid=(SPIRAL,),
        in_specs=[
            pl.BlockSpec((N_PTS, IN_C), lambda s: (0, 0)),
            pl.BlockSpec((1, IN_C, OUT_C), lambda s: (s, 0, 0)),
        ],
        out_specs=pl.BlockSpec((1, N_PTS, OUT_C), lambda s: (s, 0, 0)),
        out_shape=jax.ShapeDtypeStruct((SPIRAL, N_PTS, OUT_C), jnp.float32),
    )(x2d, wmat3)


_mesh = plsc.VectorSubcoreMesh(core_axis_name="c", subcore_axis_name="s")


@functools.partial(
    pl.kernel,
    out_type=jax.ShapeDtypeStruct((N_PTS, OUT_C), jnp.float32),
    mesh=_mesh,
    scratch_types=[
        pltpu.VMEM((ROWS,), jnp.int32),
        pltpu.VMEM((NG, 128), jnp.int32),
        pltpu.VMEM((ROWS, OUT_C), jnp.float32),
        pltpu.VMEM((P, OUT_C), jnp.float32),
        pltpu.VMEM((OUT_C,), jnp.float32),
        pltpu.SemaphoreType.DMA,
    ],
)
def _sc_gather(y_hbm, adj_hbm, b_hbm, out_hbm,
               adj_v, idx_v, rows_v, out_v, bias_v, sem):
    wid = lax.axis_index("s") * NC + lax.axis_index("c")
    pltpu.sync_copy(b_hbm, bias_v)
    n_iter = (NCHUNK - wid + NW - 1) // NW

    def chunk_body(i, carry):
        c = wid + i * NW
        pltpu.sync_copy(adj_hbm.at[pl.ds(c * ROWS, ROWS)], adj_v)
        for j in range(NG):
            for v in range(128 // L):
                f0 = j * 128 + v * L
                a = adj_v[pl.ds(f0, L)]
                svec = (lax.iota(jnp.int32, L) + (f0 % SPIRAL)) * N_PTS
                idx_v[j, pl.ds(v * L, L)] = a + svec
        cps = [
            pltpu.async_copy(y_hbm.at[idx_v.at[j]],
                             rows_v.at[pl.ds(j * 128, 128)], sem)
            for j in range(NG)
        ]
        for cp in cps:
            cp.wait()
        for p in range(P):
            accs = tuple(bias_v[pl.ds(v * L, L)] for v in range(VPR))

            def s_body(s, acc):
                r = p * SPIRAL + s
                return tuple(a + rows_v[r, pl.ds(v * L, L)]
                             for v, a in enumerate(acc))

            accs = lax.fori_loop(0, SPIRAL, s_body, accs)
            for v in range(VPR):
                z = accs[v]
                y = jnp.where(z > 0.0,
                              z, jnp.exp(jnp.minimum(z, 0.0)) - 1.0)
                out_v[p, pl.ds(v * L, L)] = y

            @pl.when(c * P + p == N_PTS - 1)
            def _():
                zero = jnp.zeros((L,), jnp.float32)
                for v in range(VPR):
                    out_v[p, pl.ds(v * L, L)] = zero
        pltpu.sync_copy(out_v, out_hbm.at[pl.ds(c * P, P)])
        return carry

    lax.fori_loop(0, n_iter, chunk_body, 0)


def kernel(x, spiral_adj, W, b):
    x2d = x.reshape(N_PTS, IN_C)
    adj = spiral_adj.reshape(N_PTS * SPIRAL).astype(jnp.int32)
    wmat3 = W.reshape(OUT_C, SPIRAL, IN_C).transpose(1, 2, 0)
    y = _matmul(x2d, wmat3)
    yr = y.reshape(SPIRAL * N_PTS, OUT_C)
    out2d = _sc_gather(yr, adj, b)
    return out2d.reshape(1, N_PTS, OUT_C)


# speedup vs baseline: 18.9049x; 1.3812x over previous
import functools

import jax
import jax.numpy as jnp
from jax import lax
from jax.experimental import pallas as pl
from jax.experimental.pallas import tpu as pltpu
from jax.experimental.pallas import tpu_sc as plsc

IN_C = 128
SPIRAL = 32
OUT_C = 128
N_PTS = 10000

_info = plsc.get_sparse_core_info()
NC = _info.num_cores
NS = _info.num_subcores
L = _info.num_lanes
NW = NC * NS

P = 8
ROWS = P * SPIRAL
NG = ROWS // 128
NCHUNK = N_PTS // P
NIT = (NCHUNK + NW - 1) // NW
VPR = OUT_C // L


def _mm_body(x_ref, w_ref, y_ref):
    y_ref[0] = jnp.dot(x_ref[...], w_ref[0],
                       preferred_element_type=jnp.float32)


def _matmul(x2d, wmat3):
    return pl.pallas_call(
        _mm_body,
        grid=(SPIRAL,),
        in_specs=[
            pl.BlockSpec((N_PTS, IN_C), lambda s: (0, 0)),
            pl.BlockSpec((1, IN_C, OUT_C), lambda s: (s, 0, 0)),
        ],
        out_specs=pl.BlockSpec((1, N_PTS, OUT_C), lambda s: (s, 0, 0)),
        out_shape=jax.ShapeDtypeStruct((SPIRAL, N_PTS, OUT_C), jnp.float32),
    )(x2d, wmat3)


_mesh = plsc.VectorSubcoreMesh(core_axis_name="c", subcore_axis_name="s")


@functools.partial(
    pl.kernel,
    out_type=jax.ShapeDtypeStruct((N_PTS, OUT_C), jnp.float32),
    mesh=_mesh,
    scratch_types=[
        pltpu.VMEM((ROWS,), jnp.int32),
        pltpu.VMEM((ROWS,), jnp.int32),
        pltpu.VMEM((NG, 128), jnp.int32),
        pltpu.VMEM((NG, 128), jnp.int32),
        pltpu.VMEM((ROWS, OUT_C), jnp.float32),
        pltpu.VMEM((ROWS, OUT_C), jnp.float32),
        pltpu.VMEM((P, OUT_C), jnp.float32),
        pltpu.VMEM((OUT_C,), jnp.float32),
        pltpu.SemaphoreType.DMA,
        pltpu.SemaphoreType.DMA,
        pltpu.SemaphoreType.DMA,
        pltpu.SemaphoreType.DMA,
    ],
)
def _sc_gather(y_hbm, adj_hbm, b_hbm, out_hbm,
               adj0, adj1, idx0, idx1, rows0, rows1, out_v, bias_v,
               sema0, sema1, semr0, semr1):
    wid = lax.axis_index("s") * NC + lax.axis_index("c")
    pltpu.sync_copy(b_hbm, bias_v)

    def chunk_of(i):
        return jnp.minimum(wid + i * NW, NCHUNK - 1)

    def adj_cp(i, adj_v, sema):
        c = chunk_of(i)
        return pltpu.make_async_copy(
            adj_hbm.at[pl.ds(c * ROWS, ROWS)], adj_v, sema)

    def gather_cps(idx_v, rows_v, semr):
        return [
            pltpu.make_async_copy(y_hbm.at[idx_v.at[j]],
                                  rows_v.at[pl.ds(j * 128, 128)], semr)
            for j in range(NG)
        ]

    def build_idx(adj_v, idx_v):
        for j in range(NG):
            for v in range(128 // L):
                f0 = j * 128 + v * L
                a = adj_v[pl.ds(f0, L)]
                svec = (lax.iota(jnp.int32, L) + (f0 % SPIRAL)) * N_PTS
                idx_v[j, pl.ds(v * L, L)] = a + svec

    def step(i, cur, nxt):
        (c_adj, c_idx, c_rows, c_sema, c_semr) = cur
        (n_adj, n_idx, n_rows, n_sema, n_semr) = nxt

        @pl.when(i + 1 < NIT)
        def _():
            adj_cp(i + 1, n_adj, n_sema).wait()
            build_idx(n_adj, n_idx)
            for cp in gather_cps(n_idx, n_rows, n_semr):
                cp.start()

        @pl.when(i + 2 < NIT)
        def _():
            adj_cp(i + 2, c_adj, c_sema).start()

        for cp in gather_cps(c_idx, c_rows, c_semr):
            cp.wait()

        c = chunk_of(i)
        for p in range(P):
            accs = tuple(bias_v[pl.ds(v * L, L)] for v in range(VPR))

            def s_body(s, acc):
                r = p * SPIRAL + s
                return tuple(a + c_rows[r, pl.ds(v * L, L)]
                             for v, a in enumerate(acc))

            accs = lax.fori_loop(0, SPIRAL, s_body, accs)
            for v in range(VPR):
                z = accs[v]
                y = jnp.where(z > 0.0,
                              z, jnp.exp(jnp.minimum(z, 0.0)) - 1.0)
                out_v[p, pl.ds(v * L, L)] = y

            @pl.when(c * P + p == N_PTS - 1)
            def _():
                zero = jnp.zeros((L,), jnp.float32)
                for v in range(VPR):
                    out_v[p, pl.ds(v * L, L)] = zero

        @pl.when(wid + i * NW < NCHUNK)
        def _():
            pltpu.sync_copy(out_v, out_hbm.at[pl.ds(c * P, P)])

    slot0 = (adj0, idx0, rows0, sema0, semr0)
    slot1 = (adj1, idx1, rows1, sema1, semr1)

    adj_cp(0, adj0, sema0).start()
    adj_cp(0, adj0, sema0).wait()
    build_idx(adj0, idx0)
    for cp in gather_cps(idx0, rows0, semr0):
        cp.start()
    adj_cp(1, adj1, sema1).start()

    def pair_body(g, carry):
        step(2 * g, slot0, slot1)
        step(2 * g + 1, slot1, slot0)
        return carry

    lax.fori_loop(0, NIT // 2, pair_body, 0)


def kernel(x, spiral_adj, W, b):
    x2d = x.reshape(N_PTS, IN_C).astype(jnp.bfloat16)
    adj = spiral_adj.reshape(N_PTS * SPIRAL).astype(jnp.int32)
    wmat3 = (W.reshape(OUT_C, SPIRAL, IN_C).transpose(1, 2, 0)
             .astype(jnp.bfloat16))
    y = _matmul(x2d, wmat3)
    yr = y.reshape(SPIRAL * N_PTS, OUT_C)
    out2d = _sc_gather(yr, adj, b)
    return out2d.reshape(1, N_PTS, OUT_C)
